# baseline (speedup 1.0000x reference)
import jax
import jax.numpy as jnp
from jax.experimental import pallas as pl
from jax.experimental.pallas import tpu as pltpu

K = 100
N = 1_000_000
BLOCK = 8192
NBLK = (N + BLOCK - 1) // BLOCK


def _matmul_kernel(img_ref, idx_ref, sims_ref):
    sims = jax.lax.dot_general(
        img_ref[...], idx_ref[...],
        dimension_numbers=(((1,), (1,)), ((), ())),
        preferred_element_type=jnp.float32,
    )
    b = pl.program_id(0)
    cols = b * BLOCK + jax.lax.broadcasted_iota(jnp.int32, sims.shape, 1)
    sims_ref[...] = jnp.where(cols < N, sims, -jnp.inf)


def kernel(image_embs, index):
    sims = pl.pallas_call(
        _matmul_kernel,
        grid=(NBLK,),
        in_specs=[
            pl.BlockSpec((32, 64), lambda b: (0, 0)),
            pl.BlockSpec((BLOCK, 64), lambda b: (b, 0)),
        ],
        out_specs=pl.BlockSpec((32, BLOCK), lambda b: (0, b)),
        out_shape=jax.ShapeDtypeStruct((32, NBLK * BLOCK), jnp.float32),
    )(image_embs, index)
    _, idx = jax.lax.top_k(sims[:, :N], K)
    return idx.astype(jnp.int32)


# speedup vs baseline: 8.8576x; 8.8576x over previous
import dataclasses
import functools

import jax
import jax.numpy as jnp
from jax import lax
from jax.experimental import pallas as pl
from jax.experimental.pallas import tpu as pltpu
from jax.experimental.pallas import tpu_sc as plsc

K = 100
N = 1_000_000
ROWS = 32
BLOCK = 8192
NBLK = (N + BLOCK - 1) // BLOCK
NPAD = NBLK * BLOCK
NSLOT = BLOCK
FOLD = 1024
CAP = 2048
SLOTCAP = 1024
NEG_INF = float("-inf")

_SC_PARAMS = pltpu.CompilerParams()
if "needs_layout_passes" in pltpu.CompilerParams.__dataclass_fields__:
    _SC_PARAMS = dataclasses.replace(
        _SC_PARAMS, needs_layout_passes=False, use_tc_tiling_on_sc=False)


def _stage_a(img_ref, idx_ref, sims_ref, smax_ref, thr_ref):
    b = pl.program_id(0)
    sims = lax.dot_general(
        img_ref[...], idx_ref[...],
        dimension_numbers=(((1,), (1,)), ((), ())),
        preferred_element_type=jnp.float32,
    )

    @pl.when(b == 0)
    def _():
        sims_ref[...] = sims
        smax_ref[...] = sims

    @pl.when(jnp.logical_and(b > 0, b < NBLK - 1))
    def _():
        sims_ref[...] = sims
        smax_ref[...] = jnp.maximum(smax_ref[...], sims)

    @pl.when(b == NBLK - 1)
    def _():
        cols = b * BLOCK + lax.broadcasted_iota(jnp.int32, sims.shape, 1)
        masked = jnp.where(cols < N, sims, NEG_INF)
        sims_ref[...] = masked
        smax_ref[...] = jnp.maximum(smax_ref[...], masked)

        rm = smax_ref[...]
        v = rm[:, :FOLD]
        for off in range(1, NSLOT // FOLD):
            v = jnp.maximum(v, rm[:, off * FOLD:(off + 1) * FOLD])

        def body(_, carry):
            v, _m = carry
            m = jnp.max(v, axis=1, keepdims=True)
            v = jnp.where(v == m, NEG_INF, v)
            return v, m

        _, t = lax.fori_loop(0, K, body, (v, jnp.zeros((ROWS, 1), jnp.float32)))
        thr_ref[...] = jnp.broadcast_to(t, (ROWS, 128))


def _stage_c(cv_ref, ci_ref, out_ref, v_ref):
    v_ref[...] = cv_ref[...]
    lanes = lax.broadcasted_iota(jnp.int32, (ROWS, 128), 1)
    big = jnp.int32(2 ** 30)

    def body(it, outb):
        v = v_ref[...]
        ii = ci_ref[...]
        m = jnp.max(v, axis=1, keepdims=True)
        sel = v == m
        mi = jnp.min(jnp.where(sel, ii, big), axis=1, keepdims=True)
        v_ref[...] = jnp.where(sel & (ii == mi), NEG_INF, v)
        return jnp.where(lanes == it, mi, outb)

    out_ref[...] = lax.fori_loop(0, K, body, jnp.zeros((ROWS, 128), jnp.int32))


def kernel(image_embs, index):
    sims, slotmax, thr = pl.pallas_call(
        _stage_a,
        grid=(NBLK,),
        in_specs=[
            pl.BlockSpec((ROWS, 64), lambda b: (0, 0)),
            pl.BlockSpec((BLOCK, 64), lambda b: (b, 0)),
        ],
        out_specs=[
            pl.BlockSpec((ROWS, BLOCK), lambda b: (0, b)),
            pl.BlockSpec((ROWS, NSLOT), lambda b: (0, 0)),
            pl.BlockSpec((ROWS, 128), lambda b: (0, 0)),
        ],
        out_shape=[
            jax.ShapeDtypeStruct((ROWS, NPAD), jnp.float32),
            jax.ShapeDtypeStruct((ROWS, NSLOT), jnp.float32),
            jax.ShapeDtypeStruct((ROWS, 128), jnp.float32),
        ],
    )(image_embs, index)

    @functools.partial(
        pl.kernel,
        out_type=[
            jax.ShapeDtypeStruct((ROWS, CAP), jnp.float32),
            jax.ShapeDtypeStruct((ROWS, CAP), jnp.int32),
        ],
        mesh=plsc.VectorSubcoreMesh(
            core_axis_name="core", subcore_axis_name="subcore"
        ),
        scratch_types=[
            pltpu.VMEM((NSLOT,), jnp.float32),
            pltpu.VMEM((128,), jnp.float32),
            pltpu.VMEM((SLOTCAP,), jnp.int32),
            pltpu.VMEM((128, 16, 16), jnp.float32),
            pltpu.VMEM((CAP,), jnp.float32),
            pltpu.VMEM((CAP,), jnp.int32),
            pltpu.SemaphoreType.DMA,
        ],
        compiler_params=_SC_PARAMS,
    )
    def sc_select(sims_hbm, smax_hbm, thr_hbm, dummy_hbm, cv_hbm, ci_hbm,
                  smax_v, thr_v, hits_v, gval_v, cv_v, ci_v, dsem):
        r = lax.axis_index("subcore") * 2 + lax.axis_index("core")
        pltpu.sync_copy(smax_hbm.at[r], smax_v)
        pltpu.sync_copy(thr_hbm.at[r], thr_v)
        tvec = thr_v[pl.ds(0, 16)]
        iota16 = lax.iota(jnp.int32, 16)

        def init_body(i, c):
            cv_v[pl.ds(i * 16, 16)] = jnp.full((16,), NEG_INF, jnp.float32)
            ci_v[pl.ds(i * 16, 16)] = jnp.zeros((16,), jnp.int32)
            return c
        lax.fori_loop(0, CAP // 16, init_body, 0)

        def scan_body(i, cnt):
            chunk = smax_v[pl.ds(i * 16, 16)]
            m = chunk >= tvec
            sid = iota16 + i * 16
            off = jnp.minimum(cnt, SLOTCAP - 16)
            plsc.store_compressed(hits_v.at[pl.ds(off, 16)], sid, mask=m)
            return cnt + jnp.sum(m.astype(jnp.int32))

        cnt = lax.fori_loop(0, NSLOT // 16, scan_body, jnp.int32(0))
        cnt = jnp.minimum(cnt, SLOTCAP)
        nchunks = (cnt + 15) >> 4
        rowbase = r * (NPAD // 16)

        def chunk_body(c, ccnt):
            svec = hits_v[pl.ds(c * 16, 16)]
            valid = (iota16 + c * 16) < jnp.broadcast_to(cnt, (16,))
            svec = jnp.where(valid, svec, 0)
            lane = svec & 15

            def fire(b, z):
                idxv = jnp.where(b < NBLK,
                                 rowbase + ((svec + b * BLOCK) >> 4), 0)
                pltpu.async_copy(sims_hbm.at[idxv], gval_v.at[b], dsem)
                return z
            lax.fori_loop(0, 128, fire, 0)
            pltpu.make_async_copy(dummy_hbm, gval_v, dsem).wait()

            def filt(b, ccnt):
                g = plsc.load_gather(gval_v.at[b], [iota16, lane])
                gi = svec + b * BLOCK
                m = jnp.logical_and(g >= tvec, valid)
                off = jnp.minimum(ccnt, CAP - 16)
                plsc.store_compressed(cv_v.at[pl.ds(off, 16)], g, mask=m)
                plsc.store_compressed(ci_v.at[pl.ds(off, 16)], gi, mask=m)
                return ccnt + jnp.sum(m.astype(jnp.int32))

            return lax.fori_loop(0, NBLK, filt, ccnt)

        lax.fori_loop(0, nchunks, chunk_body, jnp.int32(0))
        pltpu.sync_copy(cv_v, cv_hbm.at[r])
        pltpu.sync_copy(ci_v, ci_hbm.at[r])

    cv, ci = sc_select(sims.reshape(ROWS * NPAD // 16, 16), slotmax, thr,
                       jnp.zeros((128, 16, 16), jnp.float32))

    idx = pl.pallas_call(
        _stage_c,
        out_shape=jax.ShapeDtypeStruct((ROWS, 128), jnp.int32),
        scratch_shapes=[pltpu.VMEM((ROWS, CAP), jnp.float32)],
    )(cv, ci)
    return idx[:, :K]


# speedup vs baseline: 8.8757x; 1.0020x over previous
import dataclasses
import functools

import jax
import jax.numpy as jnp
from jax import lax
from jax.experimental import pallas as pl
from jax.experimental.pallas import tpu as pltpu
from jax.experimental.pallas import tpu_sc as plsc

K = 100
N = 1_000_000
ROWS = 32
BLOCK = 8192
NBLK = (N + BLOCK - 1) // BLOCK
NPAD = NBLK * BLOCK
NSLOT = BLOCK
FOLD = 1024
CAP = 2048
SLOTCAP = 1024
NEG_INF = float("-inf")

_SC_PARAMS = pltpu.CompilerParams()
if "needs_layout_passes" in pltpu.CompilerParams.__dataclass_fields__:
    _SC_PARAMS = dataclasses.replace(
        _SC_PARAMS, needs_layout_passes=False, use_tc_tiling_on_sc=False)


def _stage_a(img_ref, idx_ref, sims_ref, smax_ref, thr_ref):
    b = pl.program_id(0)
    sims = lax.dot_general(
        img_ref[...], idx_ref[...],
        dimension_numbers=(((1,), (1,)), ((), ())),
        preferred_element_type=jnp.float32,
    )

    @pl.when(b == 0)
    def _():
        sims_ref[...] = sims
        smax_ref[...] = sims

    @pl.when(jnp.logical_and(b > 0, b < NBLK - 1))
    def _():
        sims_ref[...] = sims
        smax_ref[...] = jnp.maximum(smax_ref[...], sims)

    @pl.when(b == NBLK - 1)
    def _():
        cols = b * BLOCK + lax.broadcasted_iota(jnp.int32, sims.shape, 1)
        masked = jnp.where(cols < N, sims, NEG_INF)
        sims_ref[...] = masked
        smax_ref[...] = jnp.maximum(smax_ref[...], masked)

        rm = smax_ref[...]
        v = rm[:, :FOLD]
        for off in range(1, NSLOT // FOLD):
            v = jnp.maximum(v, rm[:, off * FOLD:(off + 1) * FOLD])

        def body(_, carry):
            v, _m = carry
            m = jnp.max(v, axis=1, keepdims=True)
            v = jnp.where(v == m, NEG_INF, v)
            return v, m

        _, t = lax.fori_loop(0, K, body, (v, jnp.zeros((ROWS, 1), jnp.float32)))
        thr_ref[...] = jnp.broadcast_to(t, (ROWS, 128))


def _stage_c(cv_ref, ci_ref, out_ref, v_ref):
    v_ref[...] = cv_ref[...]
    lanes = lax.broadcasted_iota(jnp.int32, (ROWS, 128), 1)
    big = jnp.int32(2 ** 30)

    def body(it, outb):
        v = v_ref[...]
        ii = ci_ref[...]
        m = jnp.max(v, axis=1, keepdims=True)
        sel = v == m
        mi = jnp.min(jnp.where(sel, ii, big), axis=1, keepdims=True)
        v_ref[...] = jnp.where(sel & (ii == mi), NEG_INF, v)
        return jnp.where(lanes == it, mi, outb)

    out_ref[...] = lax.fori_loop(0, K, body, jnp.zeros((ROWS, 128), jnp.int32))


def kernel(image_embs, index):
    sims, slotmax, thr = pl.pallas_call(
        _stage_a,
        grid=(NBLK,),
        in_specs=[
            pl.BlockSpec((ROWS, 64), lambda b: (0, 0)),
            pl.BlockSpec((BLOCK, 64), lambda b: (b, 0)),
        ],
        out_specs=[
            pl.BlockSpec((ROWS, BLOCK), lambda b: (0, b)),
            pl.BlockSpec((ROWS, NSLOT), lambda b: (0, 0)),
            pl.BlockSpec((ROWS, 128), lambda b: (0, 0)),
        ],
        out_shape=[
            jax.ShapeDtypeStruct((ROWS, NPAD), jnp.float32),
            jax.ShapeDtypeStruct((ROWS, NSLOT), jnp.float32),
            jax.ShapeDtypeStruct((ROWS, 128), jnp.float32),
        ],
    )(image_embs, index)

    @functools.partial(
        pl.kernel,
        out_type=[
            jax.ShapeDtypeStruct((ROWS, CAP), jnp.float32),
            jax.ShapeDtypeStruct((ROWS, CAP), jnp.int32),
        ],
        mesh=plsc.VectorSubcoreMesh(
            core_axis_name="core", subcore_axis_name="subcore"
        ),
        scratch_types=[
            pltpu.VMEM((NSLOT,), jnp.float32),
            pltpu.VMEM((128,), jnp.float32),
            pltpu.VMEM((SLOTCAP,), jnp.int32),
            pltpu.VMEM((16, 128), jnp.int32),
            pltpu.VMEM((16, 128, 16), jnp.float32),
            pltpu.VMEM((CAP,), jnp.float32),
            pltpu.VMEM((CAP,), jnp.int32),
            pltpu.SemaphoreType.DMA,
        ],
        compiler_params=_SC_PARAMS,
    )
    def sc_select(sims_hbm, smax_hbm, thr_hbm, dummy_hbm, cv_hbm, ci_hbm,
                  smax_v, thr_v, hits_v, gidx_v, gval_v, cv_v, ci_v, dsem):
        r = lax.axis_index("subcore") * 2 + lax.axis_index("core")
        pltpu.sync_copy(smax_hbm.at[r], smax_v)
        pltpu.sync_copy(thr_hbm.at[r], thr_v)
        tvec = thr_v[pl.ds(0, 16)]
        iota16 = lax.iota(jnp.int32, 16)

        def init_body(i, c):
            cv_v[pl.ds(i * 16, 16)] = jnp.full((16,), NEG_INF, jnp.float32)
            ci_v[pl.ds(i * 16, 16)] = jnp.zeros((16,), jnp.int32)
            return c
        lax.fori_loop(0, CAP // 16, init_body, 0)

        def scan_body(i, cnt):
            chunk = smax_v[pl.ds(i * 16, 16)]
            m = chunk >= tvec
            sid = iota16 + i * 16
            off = jnp.minimum(cnt, SLOTCAP - 16)
            plsc.store_compressed(hits_v.at[pl.ds(off, 16)], sid, mask=m)
            return cnt + jnp.sum(m.astype(jnp.int32))

        cnt = lax.fori_loop(0, NSLOT // 16, scan_body, jnp.int32(0))
        cnt = jnp.minimum(cnt, SLOTCAP)
        nchunks = (cnt + 15) >> 4
        rowbase = r * (NPAD // 16)

        def chunk_body(c, ccnt):
            svec = hits_v[pl.ds(c * 16, 16)]
            valid = (iota16 + c * 16) < jnp.broadcast_to(cnt, (16,))
            svec = jnp.where(valid, svec, 0)
            lane = svec & 15

            def fire(bg, z):
                for bl in range(8):
                    b = bg * 8 + bl
                    idxv = jnp.where(b < NBLK,
                                     rowbase + ((svec + b * BLOCK) >> 4), 0)
                    gidx_v[bg, pl.ds(bl * 16, 16)] = idxv
                pltpu.async_copy(sims_hbm.at[gidx_v.at[bg]],
                                 gval_v.at[bg], dsem)
                return z
            lax.fori_loop(0, 16, fire, 0)
            pltpu.make_async_copy(dummy_hbm, gval_v, dsem).wait()

            def filt(b, ccnt):
                g = plsc.load_gather(gval_v.at[b >> 3],
                                     [((b & 7) << 4) + iota16, lane])
                gi = svec + b * BLOCK
                m = jnp.logical_and(g >= tvec, valid)
                off = jnp.minimum(ccnt, CAP - 16)
                plsc.store_compressed(cv_v.at[pl.ds(off, 16)], g, mask=m)
                plsc.store_compressed(ci_v.at[pl.ds(off, 16)], gi, mask=m)
                return ccnt + jnp.sum(m.astype(jnp.int32))

            return lax.fori_loop(0, NBLK, filt, ccnt)

        lax.fori_loop(0, nchunks, chunk_body, jnp.int32(0))
        pltpu.sync_copy(cv_v, cv_hbm.at[r])
        pltpu.sync_copy(ci_v, ci_hbm.at[r])

    cv, ci = sc_select(sims.reshape(ROWS * NPAD // 16, 16), slotmax, thr,
                       jnp.zeros((16, 128, 16), jnp.float32))

    idx = pl.pallas_call(
        _stage_c,
        out_shape=jax.ShapeDtypeStruct((ROWS, 128), jnp.int32),
        scratch_shapes=[pltpu.VMEM((ROWS, CAP), jnp.float32)],
    )(cv, ci)
    return idx[:, :K]
